# baseline (speedup 1.0000x reference)
import functools

import jax
import jax.numpy as jnp
from jax import lax
from jax.experimental import pallas as pl
from jax.experimental.pallas import tpu as pltpu
from jax.experimental.pallas import tpu_sc as plsc

_BATCH = 16384
_EMB = 64
_NC = 2
_NS = 16
_NW = _NC * _NS
_BPW = _BATCH // _NW

_sc_mesh = plsc.VectorSubcoreMesh(core_axis_name="c", subcore_axis_name="s")


def _gather_rows(tbl, ids_vmem, rows_v, sem):
    def issue(g, carry):
        ids16 = ids_vmem[pl.ds(g * 16, 16)]
        for k in range(16):
            rid = ids16[k]
            pltpu.async_copy(tbl.at[rid], rows_v.at[g * 16 + k], sem)
        return carry

    lax.fori_loop(0, _BPW // 16, issue, 0)
    pltpu.make_async_copy(tbl.at[pl.ds(0, _BPW)], rows_v, sem).wait()


@functools.partial(
    pl.kernel,
    mesh=_sc_mesh,
    out_type=[
        jax.ShapeDtypeStruct((_BATCH, _EMB), jnp.float32),
        jax.ShapeDtypeStruct((_BATCH, _EMB), jnp.float32),
    ],
    scratch_types=[
        pltpu.VMEM((_BPW,), jnp.int32),
        pltpu.VMEM((_BPW,), jnp.int32),
        pltpu.VMEM((_BPW, _EMB), jnp.float32),
        pltpu.SemaphoreType.DMA,
    ],
    compiler_params=pltpu.CompilerParams(use_tc_tiling_on_sc=True),
)
def _sc_gather(u_ids_hbm, i_ids_hbm, user_hbm, item_hbm, u_out, i_out,
               uidx_v, iidx_v, rows_v, sem):
    wid = lax.axis_index("s") * _NC + lax.axis_index("c")
    base = wid * _BPW
    pltpu.sync_copy(u_ids_hbm.at[pl.ds(base, _BPW)], uidx_v)
    pltpu.sync_copy(i_ids_hbm.at[pl.ds(base, _BPW)], iidx_v)
    _gather_rows(user_hbm, uidx_v, rows_v, sem)
    pltpu.sync_copy(rows_v, u_out.at[pl.ds(base, _BPW)])
    _gather_rows(item_hbm, iidx_v, rows_v, sem)
    pltpu.sync_copy(rows_v, i_out.at[pl.ds(base, _BPW)])


_TC_BLK = 2048


def _tc_body(u_ref, i_ref, rel_ref, nrm_ref, out_ref):
    u = u_ref[...]
    i = i_ref[...]
    rel = rel_ref[...]
    nrm = nrm_ref[...]
    s = u + i
    pre = lax.dot_general(s, rel, (((1,), (1,)), ((), ())),
                          preferred_element_type=jnp.float32,
                          precision=lax.Precision.HIGHEST) * 0.5
    r = lax.dot_general(pre, rel, (((1,), (0,)), ((), ())),
                        preferred_element_type=jnp.float32,
                        precision=lax.Precision.HIGHEST)
    n = lax.dot_general(pre, nrm, (((1,), (0,)), ((), ())),
                        preferred_element_type=jnp.float32,
                        precision=lax.Precision.HIGHEST)
    d = u - i
    t = d - jnp.sum(d * n, axis=1, keepdims=True) * n + r
    out_ref[...] = jnp.sum(jnp.abs(t), axis=1, keepdims=True)


_tc_score = pl.pallas_call(
    _tc_body,
    grid=(_BATCH // _TC_BLK,),
    in_specs=[
        pl.BlockSpec((_TC_BLK, _EMB), lambda b: (b, 0)),
        pl.BlockSpec((_TC_BLK, _EMB), lambda b: (b, 0)),
        pl.BlockSpec((_EMB, _EMB), lambda b: (0, 0)),
        pl.BlockSpec((_EMB, _EMB), lambda b: (0, 0)),
    ],
    out_specs=pl.BlockSpec((_TC_BLK, 1), lambda b: (b, 0)),
    out_shape=jax.ShapeDtypeStruct((_BATCH, 1), jnp.float32),
)


def kernel(u_ids, i_ids, user_emb, item_emb, rel_emb, norm_emb):
    u_e, i_e = _sc_gather(u_ids.astype(jnp.int32), i_ids.astype(jnp.int32),
                          user_emb, item_emb)
    score = _tc_score(u_e, i_e, rel_emb, norm_emb)
    return score[:, 0]


# speedup vs baseline: 1.0568x; 1.0568x over previous
import functools

import jax
import jax.numpy as jnp
from jax import lax
from jax.experimental import pallas as pl
from jax.experimental.pallas import tpu as pltpu
from jax.experimental.pallas import tpu_sc as plsc

_BATCH = 16384
_EMB = 64
_NC = 2
_NS = 16
_NW = _NC * _NS
_BPW = _BATCH // _NW

_sc_mesh = plsc.VectorSubcoreMesh(core_axis_name="c", subcore_axis_name="s")


def _gather_rows(tbl, ids_vmem, rows_v, sem):
    def issue(g, carry):
        ids16 = ids_vmem[pl.ds(g * 16, 16)]
        for k in range(16):
            rid = ids16[k]
            pltpu.async_copy(tbl.at[rid], rows_v.at[g * 16 + k], sem)
        return carry

    lax.fori_loop(0, _BPW // 16, issue, 0)
    pltpu.make_async_copy(tbl.at[pl.ds(0, _BPW)], rows_v, sem).wait()


@functools.partial(
    pl.kernel,
    mesh=_sc_mesh,
    out_type=[
        jax.ShapeDtypeStruct((_BATCH, _EMB), jnp.float32),
        jax.ShapeDtypeStruct((_BATCH, _EMB), jnp.float32),
    ],
    scratch_types=[
        pltpu.VMEM((_BPW,), jnp.int32),
        pltpu.VMEM((_BPW,), jnp.int32),
        pltpu.VMEM((_BPW, _EMB), jnp.float32),
        pltpu.SemaphoreType.DMA,
    ],
    compiler_params=pltpu.CompilerParams(use_tc_tiling_on_sc=True),
)
def _sc_gather(u_ids_hbm, i_ids_hbm, user_hbm, item_hbm, u_out, i_out,
               uidx_v, iidx_v, rows_v, sem):
    wid = lax.axis_index("s") * _NC + lax.axis_index("c")
    base = wid * _BPW
    pltpu.sync_copy(u_ids_hbm.at[pl.ds(base, _BPW)], uidx_v)
    pltpu.sync_copy(i_ids_hbm.at[pl.ds(base, _BPW)], iidx_v)
    _gather_rows(user_hbm, uidx_v, rows_v, sem)
    pltpu.sync_copy(rows_v, u_out.at[pl.ds(base, _BPW)])
    _gather_rows(item_hbm, iidx_v, rows_v, sem)
    pltpu.sync_copy(rows_v, i_out.at[pl.ds(base, _BPW)])


_TC_BLK = 2048


def _tc_body(u_ref, i_ref, rel_ref, nrm_ref, out_ref):
    u = u_ref[...]
    i = i_ref[...]
    rel = rel_ref[...]
    nrm = nrm_ref[...]
    s = u + i
    pre = lax.dot_general(s, rel, (((1,), (1,)), ((), ())),
                          preferred_element_type=jnp.float32) * 0.5
    r = lax.dot_general(pre, rel, (((1,), (0,)), ((), ())),
                        preferred_element_type=jnp.float32)
    n = lax.dot_general(pre, nrm, (((1,), (0,)), ((), ())),
                        preferred_element_type=jnp.float32)
    d = u - i
    t = d - jnp.sum(d * n, axis=1, keepdims=True) * n + r
    out_ref[...] = jnp.sum(jnp.abs(t), axis=1, keepdims=True)


_tc_score = pl.pallas_call(
    _tc_body,
    grid=(_BATCH // _TC_BLK,),
    in_specs=[
        pl.BlockSpec((_TC_BLK, _EMB), lambda b: (b, 0)),
        pl.BlockSpec((_TC_BLK, _EMB), lambda b: (b, 0)),
        pl.BlockSpec((_EMB, _EMB), lambda b: (0, 0)),
        pl.BlockSpec((_EMB, _EMB), lambda b: (0, 0)),
    ],
    out_specs=pl.BlockSpec((_TC_BLK, 1), lambda b: (b, 0)),
    out_shape=jax.ShapeDtypeStruct((_BATCH, 1), jnp.float32),
)


def kernel(u_ids, i_ids, user_emb, item_emb, rel_emb, norm_emb):
    u_e, i_e = _sc_gather(u_ids.astype(jnp.int32), i_ids.astype(jnp.int32),
                          user_emb, item_emb)
    score = _tc_score(u_e, i_e, rel_emb, norm_emb)
    return score[:, 0]


# speedup vs baseline: 1.0660x; 1.0086x over previous
import functools

import jax
import jax.numpy as jnp
from jax import lax
from jax.experimental import pallas as pl
from jax.experimental.pallas import tpu as pltpu
from jax.experimental.pallas import tpu_sc as plsc

_BATCH = 16384
_EMB = 64
_NC = 2
_NS = 16
_NW = _NC * _NS
_BPW = _BATCH // _NW

_sc_mesh = plsc.VectorSubcoreMesh(core_axis_name="c", subcore_axis_name="s")


def _gather_rows(tbl, ids_vmem, rows_v, sem):
    def issue(g, carry):
        ids16 = ids_vmem[pl.ds(g * 16, 16)]
        for k in range(16):
            rid = ids16[k]
            pltpu.async_copy(tbl.at[rid], rows_v.at[g * 16 + k], sem)
        return carry

    lax.fori_loop(0, _BPW // 16, issue, 0)
    pltpu.make_async_copy(tbl.at[pl.ds(0, _BPW)], rows_v, sem).wait()


@functools.partial(
    pl.kernel,
    mesh=_sc_mesh,
    out_type=jax.ShapeDtypeStruct((_BATCH, _EMB), jnp.float32),
    scratch_types=[
        pltpu.VMEM((_BPW,), jnp.int32),
        pltpu.VMEM((_BPW, _EMB), jnp.float32),
        pltpu.SemaphoreType.DMA,
    ],
    compiler_params=pltpu.CompilerParams(use_tc_tiling_on_sc=True),
)
def _sc_gather_one(ids_hbm, tbl_hbm, out, idx_v, rows_v, sem):
    wid = lax.axis_index("s") * _NC + lax.axis_index("c")
    base = wid * _BPW
    pltpu.sync_copy(ids_hbm.at[pl.ds(base, _BPW)], idx_v)
    _gather_rows(tbl_hbm, idx_v, rows_v, sem)
    pltpu.sync_copy(rows_v, out.at[pl.ds(base, _BPW)])


_TC_BLK = 2048


def _tc_body(u_ref, i_ref, rel_ref, nrm_ref, out_ref):
    u = u_ref[...]
    i = i_ref[...]
    rel = rel_ref[...]
    nrm = nrm_ref[...]
    s = u + i
    pre = lax.dot_general(s, rel, (((1,), (1,)), ((), ())),
                          preferred_element_type=jnp.float32) * 0.5
    r = lax.dot_general(pre, rel, (((1,), (0,)), ((), ())),
                        preferred_element_type=jnp.float32)
    n = lax.dot_general(pre, nrm, (((1,), (0,)), ((), ())),
                        preferred_element_type=jnp.float32)
    d = u - i
    t = d - jnp.sum(d * n, axis=1, keepdims=True) * n + r
    out_ref[...] = jnp.sum(jnp.abs(t), axis=1, keepdims=True)


_tc_score = pl.pallas_call(
    _tc_body,
    grid=(_BATCH // _TC_BLK,),
    in_specs=[
        pl.BlockSpec((_TC_BLK, _EMB), lambda b: (b, 0)),
        pl.BlockSpec((_TC_BLK, _EMB), lambda b: (b, 0)),
        pl.BlockSpec((_EMB, _EMB), lambda b: (0, 0)),
        pl.BlockSpec((_EMB, _EMB), lambda b: (0, 0)),
    ],
    out_specs=pl.BlockSpec((_TC_BLK, 1), lambda b: (b, 0)),
    out_shape=jax.ShapeDtypeStruct((_BATCH, 1), jnp.float32),
)


def kernel(u_ids, i_ids, user_emb, item_emb, rel_emb, norm_emb):
    u_e = _sc_gather_one(u_ids.astype(jnp.int32), user_emb)
    i_e = _sc_gather_one(i_ids.astype(jnp.int32), item_emb)
    score = _tc_score(u_e, i_e, rel_emb, norm_emb)
    return score[:, 0]
